# Initial kernel scaffold; baseline (speedup 1.0000x reference)
import jax
import jax.numpy as jnp
from jax.experimental import pallas as pl


def kernel(inputs, embeddings):
    raise NotImplementedError("write your pallas kernel here")


# speedup vs baseline: 1.3445x; 1.3445x over previous
import functools

import jax
import jax.numpy as jnp
from jax import lax
from jax.experimental import pallas as pl
from jax.experimental.pallas import tpu as pltpu
from jax.experimental.pallas import tpu_sc as plsc

VOCAB = 100000
EMBED_DIM = 64
N_FIELDS = 26
BATCH = 4096

_info = plsc.get_sparse_core_info()
NC, NS = _info.num_cores, _info.num_subcores
NW = NC * NS
BT = BATCH // NW


def _body(inputs_hbm, table_hbm, *refs):
    outs = refs[:N_FIELDS]
    stacked = refs[N_FIELDS]
    idx_v, rows_v, sem = refs[N_FIELDS + 1:]
    wid = lax.axis_index("s") * NC + lax.axis_index("c")
    base = wid * BT
    for f in range(N_FIELDS):
        pltpu.sync_copy(inputs_hbm.at[f, pl.ds(base, BT)], idx_v)
        pltpu.async_copy(table_hbm.at[idx_v], rows_v, sem).wait()
        pltpu.sync_copy(rows_v, outs[f].at[pl.ds(base, BT)])
        pltpu.sync_copy(rows_v, stacked.at[pl.ds(base, BT), f])


@jax.jit
def _run(inputs, embeddings):
    mesh = plsc.VectorSubcoreMesh(core_axis_name="c", subcore_axis_name="s")
    out_type = (
        [jax.ShapeDtypeStruct((BATCH, EMBED_DIM), jnp.float32)] * N_FIELDS
        + [jax.ShapeDtypeStruct((BATCH, N_FIELDS, EMBED_DIM), jnp.float32)]
    )
    fn = pl.kernel(
        _body,
        mesh=mesh,
        out_type=out_type,
        scratch_types=[
            pltpu.VMEM((BT,), jnp.int32),
            pltpu.VMEM((BT, EMBED_DIM), jnp.float32),
            pltpu.SemaphoreType.DMA,
        ],
        compiler_params=pltpu.CompilerParams(use_tc_tiling_on_sc=False),
    )
    return fn(inputs, embeddings)


def kernel(inputs, embeddings):
    res = _run(inputs, embeddings)
    return (tuple(res[:N_FIELDS]), res[N_FIELDS])


# speedup vs baseline: 1.4781x; 1.0993x over previous
import functools

import jax
import jax.numpy as jnp
from jax import lax
from jax.experimental import pallas as pl
from jax.experimental.pallas import tpu as pltpu
from jax.experimental.pallas import tpu_sc as plsc

VOCAB = 100000
EMBED_DIM = 64
N_FIELDS = 26
BATCH = 4096

_info = plsc.get_sparse_core_info()
NC, NS = _info.num_cores, _info.num_subcores
NW = NC * NS
BT = BATCH // NW


def _body(inputs_hbm, table_hbm, *refs):
    outs = refs[:N_FIELDS]
    stacked = refs[N_FIELDS]
    idx_v, rows_v, gsem, wsem = refs[N_FIELDS + 1:]
    wid = lax.axis_index("s") * NC + lax.axis_index("c")
    base = wid * BT
    pltpu.sync_copy(inputs_hbm.at[:, pl.ds(base, BT)], idx_v)
    gathers = [None] * N_FIELDS
    writes = [None] * N_FIELDS
    gathers[0] = pltpu.async_copy(table_hbm.at[idx_v.at[0]], rows_v.at[0], gsem)
    for f in range(N_FIELDS):
        b = f & 1
        gathers[f].wait()
        if f >= 1:
            for w in writes[f - 1]:
                w.wait()
        if f + 1 < N_FIELDS:
            gathers[f + 1] = pltpu.async_copy(
                table_hbm.at[idx_v.at[f + 1]], rows_v.at[1 - b], gsem)
        writes[f] = (
            pltpu.async_copy(rows_v.at[b], outs[f].at[pl.ds(base, BT)], wsem),
            pltpu.async_copy(rows_v.at[b], stacked.at[pl.ds(base, BT), f], wsem),
        )
    for w in writes[N_FIELDS - 1]:
        w.wait()


@jax.jit
def _run(inputs, embeddings):
    mesh = plsc.VectorSubcoreMesh(core_axis_name="c", subcore_axis_name="s")
    out_type = (
        [jax.ShapeDtypeStruct((BATCH, EMBED_DIM), jnp.float32)] * N_FIELDS
        + [jax.ShapeDtypeStruct((BATCH, N_FIELDS, EMBED_DIM), jnp.float32)]
    )
    fn = pl.kernel(
        _body,
        mesh=mesh,
        out_type=out_type,
        scratch_types=[
            pltpu.VMEM((N_FIELDS, BT), jnp.int32),
            pltpu.VMEM((2, BT, EMBED_DIM), jnp.float32),
            pltpu.SemaphoreType.DMA,
            pltpu.SemaphoreType.DMA,
        ],
        compiler_params=pltpu.CompilerParams(use_tc_tiling_on_sc=False),
    )
    return fn(inputs, embeddings)


def kernel(inputs, embeddings):
    res = _run(inputs, embeddings)
    return (tuple(res[:N_FIELDS]), res[N_FIELDS])


# speedup vs baseline: 1.5241x; 1.0311x over previous
import functools

import jax
import jax.numpy as jnp
from jax import lax
from jax.experimental import pallas as pl
from jax.experimental.pallas import tpu as pltpu
from jax.experimental.pallas import tpu_sc as plsc

VOCAB = 100000
EMBED_DIM = 64
N_FIELDS = 26
BATCH = 4096

_info = plsc.get_sparse_core_info()
NC, NS = _info.num_cores, _info.num_subcores
NW = NC * NS
BT = BATCH // NW
NB = 8
GDEPTH = NB - 2


def _body(inputs_hbm, table_hbm, *refs):
    outs = refs[:N_FIELDS]
    stacked = refs[N_FIELDS]
    idx_v, rows_v, gsem, wsem = refs[N_FIELDS + 1:]
    wid = lax.axis_index("s") * NC + lax.axis_index("c")
    base = wid * BT
    pltpu.sync_copy(inputs_hbm.at[:, pl.ds(base, BT)], idx_v)

    def gather(f):
        return pltpu.async_copy(
            table_hbm.at[idx_v.at[f]], rows_v.at[f % NB], gsem)

    gathers = [None] * N_FIELDS
    writes = [None] * N_FIELDS
    for g in range(min(GDEPTH, N_FIELDS)):
        gathers[g] = gather(g)
    waited_w = 0
    for f in range(N_FIELDS):
        g = f + GDEPTH
        if g < N_FIELDS:
            if g - NB >= 0:
                for w in writes[g - NB]:
                    w.wait()
                waited_w = g - NB + 1
            gathers[g] = gather(g)
        gathers[f].wait()
        b = f % NB
        writes[f] = (
            pltpu.async_copy(rows_v.at[b], outs[f].at[pl.ds(base, BT)], wsem),
            pltpu.async_copy(rows_v.at[b], stacked.at[pl.ds(base, BT), f], wsem),
        )
    for k in range(waited_w, N_FIELDS):
        for w in writes[k]:
            w.wait()


@jax.jit
def _run(inputs, embeddings):
    mesh = plsc.VectorSubcoreMesh(core_axis_name="c", subcore_axis_name="s")
    out_type = (
        [jax.ShapeDtypeStruct((BATCH, EMBED_DIM), jnp.float32)] * N_FIELDS
        + [jax.ShapeDtypeStruct((BATCH, N_FIELDS, EMBED_DIM), jnp.float32)]
    )
    fn = pl.kernel(
        _body,
        mesh=mesh,
        out_type=out_type,
        scratch_types=[
            pltpu.VMEM((N_FIELDS, BT), jnp.int32),
            pltpu.VMEM((NB, BT, EMBED_DIM), jnp.float32),
            pltpu.SemaphoreType.DMA,
            pltpu.SemaphoreType.DMA,
        ],
        compiler_params=pltpu.CompilerParams(use_tc_tiling_on_sc=False),
    )
    return fn(inputs, embeddings)


def kernel(inputs, embeddings):
    res = _run(inputs, embeddings)
    return (tuple(res[:N_FIELDS]), res[N_FIELDS])


# speedup vs baseline: 2.5125x; 1.6485x over previous
import jax
import jax.numpy as jnp
from jax import lax
from jax.experimental import pallas as pl
from jax.experimental.pallas import tpu as pltpu
from jax.experimental.pallas import tpu_sc as plsc

VOCAB = 100000
EMBED_DIM = 64
N_FIELDS = 26
BATCH = 4096

_info = plsc.get_sparse_core_info()
NC, NS, NL = _info.num_cores, _info.num_subcores, _info.num_lanes
NW = NC * NS
E_PER = EMBED_DIM // NW
N_VEC = BATCH // NL
UNROLL = 8


def _body(inputs_hbm, embT_hbm, *refs):
    outsT = refs[:N_FIELDS]
    stackedT = refs[N_FIELDS]
    row_v, idx_v, out_v, isem, wsem = refs[N_FIELDS + 1:]
    wid = lax.axis_index("s") * NC + lax.axis_index("c")
    e0 = wid * E_PER

    def gather_field(bi):
        def body(i, _):
            for k in range(UNROLL):
                sl = pl.ds((i * UNROLL + k) * NL, NL)
                out_v[bi, sl] = plsc.load_gather(row_v, [idx_v[bi, sl]])
            return 0

        lax.fori_loop(0, N_VEC // UNROLL, body, 0, unroll=False)

    jobs = [(j, f) for j in range(E_PER) for f in range(N_FIELDS)]
    idx_cp = [None] * len(jobs)
    writes = [None] * len(jobs)
    idx_cp[0] = pltpu.async_copy(inputs_hbm.at[jobs[0][1]], idx_v.at[0], isem)
    for n, (j, f) in enumerate(jobs):
        b = n & 1
        if f == 0:
            pltpu.sync_copy(embT_hbm.at[e0 + j], row_v)
        if n + 1 < len(jobs):
            idx_cp[n + 1] = pltpu.async_copy(
                inputs_hbm.at[jobs[n + 1][1]], idx_v.at[1 - b], isem)
        idx_cp[n].wait()
        if n >= 2:
            for w in writes[n - 2]:
                w.wait()
        gather_field(b)
        writes[n] = (
            pltpu.async_copy(out_v.at[b], outsT[f].at[e0 + j], wsem),
            pltpu.async_copy(out_v.at[b], stackedT.at[f, e0 + j], wsem),
        )
    for n in (len(jobs) - 2, len(jobs) - 1):
        for w in writes[n]:
            w.wait()


@jax.jit
def _run(inputs, embT):
    mesh = plsc.VectorSubcoreMesh(core_axis_name="c", subcore_axis_name="s")
    out_type = (
        [jax.ShapeDtypeStruct((EMBED_DIM, BATCH), jnp.float32)] * N_FIELDS
        + [jax.ShapeDtypeStruct((N_FIELDS, EMBED_DIM, BATCH), jnp.float32)]
    )
    fn = pl.kernel(
        _body,
        mesh=mesh,
        out_type=out_type,
        scratch_types=[
            pltpu.VMEM((VOCAB,), jnp.float32),
            pltpu.VMEM((2, BATCH), jnp.int32),
            pltpu.VMEM((2, BATCH), jnp.float32),
            pltpu.SemaphoreType.DMA,
            pltpu.SemaphoreType.DMA,
        ],
        compiler_params=pltpu.CompilerParams(needs_layout_passes=False),
    )
    return fn(inputs, embT)


def kernel(inputs, embeddings):
    res = _run(inputs, embeddings.T)
    emb_list = tuple(r.T for r in res[:N_FIELDS])
    emb_tensor = jnp.transpose(res[N_FIELDS], (2, 0, 1))
    return (emb_list, emb_tensor)
